# Initial kernel scaffold; baseline (speedup 1.0000x reference)
import jax
import jax.numpy as jnp
from jax.experimental import pallas as pl


def kernel(x, bridge, conv0_w, conv0_b, bn0_gamma, bn0_beta, conv1_w, conv1_b, bn1_gamma, bn1_beta, conv2_w, conv2_b, bn2_gamma, bn2_beta):
    raise NotImplementedError("write your pallas kernel here")


# speedup vs baseline: 1.6854x; 1.6854x over previous
import functools

import numpy as np
import jax
import jax.numpy as jnp
from jax.experimental import pallas as pl
from jax.experimental.pallas import tpu as pltpu

_VMEM_LIMIT_BYTES = 64 * 1024 * 1024
_CDT = jnp.bfloat16


def _bilin_matrix(li, lo):
    A = np.zeros((lo, li), dtype=np.float32)
    if li == 1:
        A[:, 0] = 1.0
        return A
    src = np.arange(lo, dtype=np.float64) * (li - 1) / (lo - 1)
    lo_idx = np.clip(np.floor(src).astype(np.int64), 0, li - 2)
    frac = src - lo_idx
    A[np.arange(lo), lo_idx] = (1.0 - frac).astype(np.float32)
    A[np.arange(lo), lo_idx + 1] = frac.astype(np.float32)
    return A


def _up_kernel(x_ref, ah_ref, awt_ref, o_ref):
    bc, h, w = x_ref.shape
    _, ho, wo = o_ref.shape
    x2 = x_ref[...].reshape(bc * h, w)
    t = jnp.dot(x2, awt_ref[...], preferred_element_type=jnp.float32)
    t3 = t.reshape(bc, h, wo)
    ah_b = jnp.broadcast_to(ah_ref[...], (bc, ho, h))
    o = jax.lax.dot_general(ah_b, t3, (((2,), (1,)), ((0,), (0,))),
                            preferred_element_type=jnp.float32)
    o_ref[...] = o.astype(o_ref.dtype)


def _upsample_x2(x_nchw, out_dtype):
    n, c, h, w = x_nchw.shape
    ho, wo = 2 * h, 2 * w
    ah = jnp.asarray(_bilin_matrix(h, ho))
    awt = jnp.asarray(_bilin_matrix(w, wo).T)
    nc = n * c
    bc = c
    xf = x_nchw.reshape(nc, h, w)
    out = pl.pallas_call(
        _up_kernel,
        out_shape=jax.ShapeDtypeStruct((nc, ho, wo), out_dtype),
        grid=(nc // bc,),
        in_specs=[
            pl.BlockSpec((bc, h, w), lambda i: (i, 0, 0)),
            pl.BlockSpec((ho, h), lambda i: (0, 0)),
            pl.BlockSpec((w, wo), lambda i: (0, 0)),
        ],
        out_specs=pl.BlockSpec((bc, ho, wo), lambda i: (i, 0, 0)),
        compiler_params=pltpu.CompilerParams(
            dimension_semantics=("parallel",),
            vmem_limit_bytes=_VMEM_LIMIT_BYTES),
    )(xf, ah, awt)
    return out.reshape(n, c, ho, wo)


def _im2col(x, th, wdim):
    cin = x.shape[-1]
    taps = [x[dy:dy + th, dx:dx + wdim, :].reshape(th * wdim, cin)
            for dy in range(3) for dx in range(3)]
    return jnp.concatenate(taps, axis=-1)


def _conv0_kernel(up_hbm, br_hbm, wu_ref, wb_ref, b_ref, y_ref, stat_ref,
                  ubuf, bbuf, sem_u, sem_b):
    n = pl.program_id(0)
    t = pl.program_id(1)
    _, th, wdim, cout = y_ref.shape
    cu = pltpu.make_async_copy(up_hbm.at[n, pl.ds(t * th, th + 2)], ubuf, sem_u)
    cb = pltpu.make_async_copy(br_hbm.at[n, pl.ds(t * th, th + 2)], bbuf, sem_b)
    cu.start()
    cb.start()
    cu.wait()
    cb.wait()
    au = _im2col(ubuf[...], th, wdim)
    ab = _im2col(bbuf[...], th, wdim)
    acc = jnp.dot(au, wu_ref[...], preferred_element_type=jnp.float32)
    acc = acc + jnp.dot(ab, wb_ref[...], preferred_element_type=jnp.float32)
    y = acc + b_ref[...]
    y_ref[0] = y.reshape(th, wdim, cout).astype(y_ref.dtype)
    s = jnp.sum(y, axis=0, keepdims=True)
    ss = jnp.sum(y * y, axis=0, keepdims=True)
    stat_ref[0, 0] = jnp.concatenate([s, ss], axis=0)


def _conv_bnr_kernel(h_total, xp_hbm, w_ref, b_ref, sc_ref, sh_ref,
                     y_ref, stat_ref, xbuf, sem):
    n = pl.program_id(0)
    t = pl.program_id(1)
    _, th, wdim, cout = y_ref.shape
    cp = pltpu.make_async_copy(xp_hbm.at[n, pl.ds(t * th, th + 2)], xbuf, sem)
    cp.start()
    cp.wait()
    x = xbuf[...].astype(jnp.float32)
    x = jnp.maximum(x * sc_ref[...] + sh_ref[...], 0.0)
    hp, wp, _ = x.shape
    row = jax.lax.broadcasted_iota(jnp.int32, (hp, wp, 1), 0) + t * th
    col = jax.lax.broadcasted_iota(jnp.int32, (hp, wp, 1), 1)
    valid = (row >= 1) & (row <= h_total) & (col >= 1) & (col <= wdim)
    x = jnp.where(valid, x, 0.0).astype(_CDT)
    a = _im2col(x, th, wdim)
    y = jnp.dot(a, w_ref[...], preferred_element_type=jnp.float32) + b_ref[...]
    y_ref[0] = y.reshape(th, wdim, cout).astype(y_ref.dtype)
    s = jnp.sum(y, axis=0, keepdims=True)
    ss = jnp.sum(y * y, axis=0, keepdims=True)
    stat_ref[0, 0] = jnp.concatenate([s, ss], axis=0)


def _bnr_out_kernel(y_ref, sc_ref, sh_ref, o_ref):
    o_ref[...] = jnp.maximum(
        y_ref[...].astype(jnp.float32) * sc_ref[...] + sh_ref[...], 0.0)


def _scale_shift(stats, gamma, beta, m, eps=1e-5):
    total = jnp.sum(stats, axis=(0, 1))
    mean = total[0] / m
    var = jnp.maximum(total[1] / m - mean * mean, 0.0)
    scale = gamma * jax.lax.rsqrt(var + eps)
    shift = beta - mean * scale
    return scale, shift


def _wmat(w_oihw):
    cout = w_oihw.shape[0]
    cin = w_oihw.shape[1]
    return jnp.transpose(w_oihw, (2, 3, 1, 0)).reshape(9 * cin, cout).astype(_CDT)


def kernel(x, bridge, conv0_w, conv0_b, bn0_gamma, bn0_beta,
           conv1_w, conv1_b, bn1_gamma, bn1_beta,
           conv2_w, conv2_b, bn2_gamma, bn2_beta):
    n, cx, h0, w0 = x.shape
    cb = bridge.shape[1]
    h, w = bridge.shape[2], bridge.shape[3]
    cout0 = conv0_w.shape[0]
    cout1 = conv1_w.shape[0]
    cout2 = conv2_w.shape[0]

    up = _upsample_x2(x, _CDT)
    dy = h - up.shape[2]
    dx = w - up.shape[3]
    if dy or dx:
        up = jnp.pad(up, ((0, 0), (0, 0),
                          (dy // 2, dy - dy // 2),
                          (dx // 2, dx - dx // 2)))
    up_p = jnp.pad(jnp.transpose(up, (0, 2, 3, 1)),
                   ((0, 0), (1, 1), (1, 1), (0, 0)))
    br_p = jnp.pad(jnp.transpose(bridge.astype(_CDT), (0, 2, 3, 1)),
                   ((0, 0), (1, 1), (1, 1), (0, 0)))

    th = 8
    while h % th:
        th //= 2
    nt = h // th
    grid = (n, nt)
    cparams = pltpu.CompilerParams(
        dimension_semantics=("parallel", "parallel"),
        vmem_limit_bytes=_VMEM_LIMIT_BYTES)

    w0u = _wmat(conv0_w[:, :cx])
    w0b = _wmat(conv0_w[:, cx:])

    y0, st0 = pl.pallas_call(
        _conv0_kernel,
        out_shape=(jax.ShapeDtypeStruct((n, h, w, cout0), _CDT),
                   jax.ShapeDtypeStruct((n, nt, 2, cout0), jnp.float32)),
        grid=grid,
        in_specs=[
            pl.BlockSpec(memory_space=pl.ANY),
            pl.BlockSpec(memory_space=pl.ANY),
            pl.BlockSpec((9 * cx, cout0), lambda i, t: (0, 0)),
            pl.BlockSpec((9 * cb, cout0), lambda i, t: (0, 0)),
            pl.BlockSpec((1, cout0), lambda i, t: (0, 0)),
        ],
        out_specs=(
            pl.BlockSpec((1, th, w, cout0), lambda i, t: (i, t, 0, 0)),
            pl.BlockSpec((1, 1, 2, cout0), lambda i, t: (i, t, 0, 0)),
        ),
        scratch_shapes=[
            pltpu.VMEM((th + 2, w + 2, cx), _CDT),
            pltpu.VMEM((th + 2, w + 2, cb), _CDT),
            pltpu.SemaphoreType.DMA(()),
            pltpu.SemaphoreType.DMA(()),
        ],
        compiler_params=cparams,
    )(up_p, br_p, w0u, w0b, conv0_b.reshape(1, cout0).astype(jnp.float32))

    m = float(n * h * w)
    sc0, sh0 = _scale_shift(st0, bn0_gamma, bn0_beta, m)

    def conv_bnr(y_prev, sc, sh, wmat, bias, cin, cout):
        yp = jnp.pad(y_prev, ((0, 0), (1, 1), (1, 1), (0, 0)))
        return pl.pallas_call(
            functools.partial(_conv_bnr_kernel, h),
            out_shape=(jax.ShapeDtypeStruct((n, h, w, cout), _CDT),
                       jax.ShapeDtypeStruct((n, nt, 2, cout), jnp.float32)),
            grid=grid,
            in_specs=[
                pl.BlockSpec(memory_space=pl.ANY),
                pl.BlockSpec((9 * cin, cout), lambda i, t: (0, 0)),
                pl.BlockSpec((1, cout), lambda i, t: (0, 0)),
                pl.BlockSpec((1, 1, cin), lambda i, t: (0, 0, 0)),
                pl.BlockSpec((1, 1, cin), lambda i, t: (0, 0, 0)),
            ],
            out_specs=(
                pl.BlockSpec((1, th, w, cout), lambda i, t: (i, t, 0, 0)),
                pl.BlockSpec((1, 1, 2, cout), lambda i, t: (i, t, 0, 0)),
            ),
            scratch_shapes=[
                pltpu.VMEM((th + 2, w + 2, cin), _CDT),
                pltpu.SemaphoreType.DMA(()),
            ],
            compiler_params=cparams,
        )(yp, wmat, bias.reshape(1, cout).astype(jnp.float32),
          sc.reshape(1, 1, cin), sh.reshape(1, 1, cin))

    y1, st1 = conv_bnr(y0, sc0, sh0, _wmat(conv1_w),
                       conv1_b, cout0, cout1)
    sc1, sh1 = _scale_shift(st1, bn1_gamma, bn1_beta, m)

    y2, st2 = conv_bnr(y1, sc1, sh1, _wmat(conv2_w),
                       conv2_b, cout1, cout2)
    sc2, sh2 = _scale_shift(st2, bn2_gamma, bn2_beta, m)

    out = pl.pallas_call(
        _bnr_out_kernel,
        out_shape=jax.ShapeDtypeStruct((n, h, w, cout2), jnp.float32),
        grid=grid,
        in_specs=[
            pl.BlockSpec((1, th, w, cout2), lambda i, t: (i, t, 0, 0)),
            pl.BlockSpec((1, 1, 1, cout2), lambda i, t: (0, 0, 0, 0)),
            pl.BlockSpec((1, 1, 1, cout2), lambda i, t: (0, 0, 0, 0)),
        ],
        out_specs=pl.BlockSpec((1, th, w, cout2), lambda i, t: (i, t, 0, 0)),
        compiler_params=cparams,
    )(y2, sc2.reshape(1, 1, 1, cout2), sh2.reshape(1, 1, 1, cout2))
    return jnp.transpose(out, (0, 3, 1, 2))


# speedup vs baseline: 1.7051x; 1.0117x over previous
import functools

import numpy as np
import jax
import jax.numpy as jnp
from jax.experimental import pallas as pl
from jax.experimental.pallas import tpu as pltpu

_VMEM_LIMIT_BYTES = 64 * 1024 * 1024
_CDT = jnp.bfloat16


def _bilin_matrix(li, lo):
    A = np.zeros((lo, li), dtype=np.float32)
    if li == 1:
        A[:, 0] = 1.0
        return A
    src = np.arange(lo, dtype=np.float64) * (li - 1) / (lo - 1)
    lo_idx = np.clip(np.floor(src).astype(np.int64), 0, li - 2)
    frac = src - lo_idx
    A[np.arange(lo), lo_idx] = (1.0 - frac).astype(np.float32)
    A[np.arange(lo), lo_idx + 1] = frac.astype(np.float32)
    return A


def _up_kernel(x_ref, ah_ref, awt_ref, o_ref):
    bc, h, w = x_ref.shape
    _, ho, wo = o_ref.shape
    x2 = x_ref[...].reshape(bc * h, w)
    t = jnp.dot(x2, awt_ref[...], preferred_element_type=jnp.float32)
    t3 = t.reshape(bc, h, wo)
    ah_b = jnp.broadcast_to(ah_ref[...], (bc, ho, h))
    o = jax.lax.dot_general(ah_b, t3, (((2,), (1,)), ((0,), (0,))),
                            preferred_element_type=jnp.float32)
    o_ref[...] = o.astype(o_ref.dtype)


def _upsample_x2(x_nchw, out_dtype):
    n, c, h, w = x_nchw.shape
    ho, wo = 2 * h, 2 * w
    ah = jnp.asarray(_bilin_matrix(h, ho))
    awt = jnp.asarray(_bilin_matrix(w, wo).T)
    nc = n * c
    bc = c
    xf = x_nchw.reshape(nc, h, w)
    out = pl.pallas_call(
        _up_kernel,
        out_shape=jax.ShapeDtypeStruct((nc, ho, wo), out_dtype),
        grid=(nc // bc,),
        in_specs=[
            pl.BlockSpec((bc, h, w), lambda i: (i, 0, 0)),
            pl.BlockSpec((ho, h), lambda i: (0, 0)),
            pl.BlockSpec((w, wo), lambda i: (0, 0)),
        ],
        out_specs=pl.BlockSpec((bc, ho, wo), lambda i: (i, 0, 0)),
        compiler_params=pltpu.CompilerParams(
            dimension_semantics=("parallel",),
            vmem_limit_bytes=_VMEM_LIMIT_BYTES),
    )(xf, ah, awt)
    return out.reshape(n, c, ho, wo)


_C0 = 8


def _im2col(x, th, wdim):
    cin = x.shape[-1]
    taps = [x[dy:dy + th, _C0 - 1 + dx:_C0 - 1 + dx + wdim, :]
            .reshape(th * wdim, cin)
            for dy in range(3) for dx in range(3)]
    return jnp.concatenate(taps, axis=-1)


def _slab_copy(hbm, buf, sem, n, t, th, nt, w):
    def mk(r0, nr, d0):
        return pltpu.make_async_copy(
            hbm.at[n, pl.ds(r0, nr)],
            buf.at[pl.ds(d0, nr), pl.ds(_C0, w)], sem)

    @pl.when(t == 0)
    def _():
        cp = mk(0, th + 1, 1)
        cp.start()
        cp.wait()

    @pl.when(jnp.logical_and(t > 0, t < nt - 1))
    def _():
        cp = mk(t * th - 1, th + 2, 0)
        cp.start()
        cp.wait()

    @pl.when(t == nt - 1)
    def _():
        cp = mk(t * th - 1, th + 1, 0)
        cp.start()
        cp.wait()


def _valid_mask(th, t, hp, wp, h_total, wdim):
    row = jax.lax.broadcasted_iota(jnp.int32, (hp, wp, 1), 0) + t * th
    col = jax.lax.broadcasted_iota(jnp.int32, (hp, wp, 1), 1)
    return (row >= 1) & (row <= h_total) & (col >= _C0) & (col < _C0 + wdim)


def _conv0_kernel(h_total, nt, up_hbm, br_hbm, wu_ref, wb_ref, b_ref,
                  y_ref, stat_ref, ubuf, bbuf, sem_u, sem_b):
    n = pl.program_id(0)
    t = pl.program_id(1)
    _, th, wdim, cout = y_ref.shape
    _slab_copy(up_hbm, ubuf, sem_u, n, t, th, nt, wdim)
    _slab_copy(br_hbm, bbuf, sem_b, n, t, th, nt, wdim)
    xu = ubuf[...]
    xb = bbuf[...]
    valid = _valid_mask(th, t, xu.shape[0], xu.shape[1], h_total, wdim)
    zero = jnp.zeros((), _CDT)
    xu = jnp.where(valid, xu, zero)
    xb = jnp.where(valid, xb, zero)
    au = _im2col(xu, th, wdim)
    ab = _im2col(xb, th, wdim)
    acc = jnp.dot(au, wu_ref[...], preferred_element_type=jnp.float32)
    acc = acc + jnp.dot(ab, wb_ref[...], preferred_element_type=jnp.float32)
    y = acc + b_ref[...]
    y_ref[0] = y.reshape(th, wdim, cout).astype(y_ref.dtype)
    s = jnp.sum(y, axis=0, keepdims=True)
    ss = jnp.sum(y * y, axis=0, keepdims=True)
    stat_ref[0, 0] = jnp.concatenate([s, ss], axis=0)


def _conv_bnr_kernel(h_total, nt, x_hbm, w_ref, b_ref, sc_ref, sh_ref,
                     y_ref, stat_ref, xbuf, sem):
    n = pl.program_id(0)
    t = pl.program_id(1)
    _, th, wdim, cout = y_ref.shape
    _slab_copy(x_hbm, xbuf, sem, n, t, th, nt, wdim)
    x = xbuf[...].astype(jnp.float32)
    x = jnp.maximum(x * sc_ref[...] + sh_ref[...], 0.0)
    valid = _valid_mask(th, t, x.shape[0], x.shape[1], h_total, wdim)
    x = jnp.where(valid, x, 0.0).astype(_CDT)
    a = _im2col(x, th, wdim)
    y = jnp.dot(a, w_ref[...], preferred_element_type=jnp.float32) + b_ref[...]
    y_ref[0] = y.reshape(th, wdim, cout).astype(y_ref.dtype)
    s = jnp.sum(y, axis=0, keepdims=True)
    ss = jnp.sum(y * y, axis=0, keepdims=True)
    stat_ref[0, 0] = jnp.concatenate([s, ss], axis=0)


def _bnr_out_kernel(y_ref, sc_ref, sh_ref, o_ref):
    o_ref[...] = jnp.maximum(
        y_ref[...].astype(jnp.float32) * sc_ref[...] + sh_ref[...], 0.0)


def _scale_shift(stats, gamma, beta, m, eps=1e-5):
    total = jnp.sum(stats, axis=(0, 1))
    mean = total[0] / m
    var = jnp.maximum(total[1] / m - mean * mean, 0.0)
    scale = gamma * jax.lax.rsqrt(var + eps)
    shift = beta - mean * scale
    return scale, shift


def _wmat(w_oihw):
    cout = w_oihw.shape[0]
    cin = w_oihw.shape[1]
    return jnp.transpose(w_oihw, (2, 3, 1, 0)).reshape(9 * cin, cout).astype(_CDT)


def kernel(x, bridge, conv0_w, conv0_b, bn0_gamma, bn0_beta,
           conv1_w, conv1_b, bn1_gamma, bn1_beta,
           conv2_w, conv2_b, bn2_gamma, bn2_beta):
    n, cx, h0, w0 = x.shape
    cb = bridge.shape[1]
    h, w = bridge.shape[2], bridge.shape[3]
    cout0 = conv0_w.shape[0]
    cout1 = conv1_w.shape[0]
    cout2 = conv2_w.shape[0]

    up = _upsample_x2(x, _CDT)
    dy = h - up.shape[2]
    dx = w - up.shape[3]
    if dy or dx:
        up = jnp.pad(up, ((0, 0), (0, 0),
                          (dy // 2, dy - dy // 2),
                          (dx // 2, dx - dx // 2)))
    up_nhwc = jnp.transpose(up, (0, 2, 3, 1))
    br_nhwc = jnp.transpose(bridge.astype(_CDT), (0, 2, 3, 1))

    th = 8
    while h % th:
        th //= 2
    nt = h // th
    grid = (n, nt)
    cparams = pltpu.CompilerParams(
        dimension_semantics=("parallel", "parallel"),
        vmem_limit_bytes=_VMEM_LIMIT_BYTES)

    w0u = _wmat(conv0_w[:, :cx])
    w0b = _wmat(conv0_w[:, cx:])

    y0, st0 = pl.pallas_call(
        functools.partial(_conv0_kernel, h, nt),
        out_shape=(jax.ShapeDtypeStruct((n, h, w, cout0), _CDT),
                   jax.ShapeDtypeStruct((n, nt, 2, cout0), jnp.float32)),
        grid=grid,
        in_specs=[
            pl.BlockSpec(memory_space=pl.ANY),
            pl.BlockSpec(memory_space=pl.ANY),
            pl.BlockSpec((9 * cx, cout0), lambda i, t: (0, 0)),
            pl.BlockSpec((9 * cb, cout0), lambda i, t: (0, 0)),
            pl.BlockSpec((1, cout0), lambda i, t: (0, 0)),
        ],
        out_specs=(
            pl.BlockSpec((1, th, w, cout0), lambda i, t: (i, t, 0, 0)),
            pl.BlockSpec((1, 1, 2, cout0), lambda i, t: (i, t, 0, 0)),
        ),
        scratch_shapes=[
            pltpu.VMEM((th + 2, _C0 + w + 8, cx), _CDT),
            pltpu.VMEM((th + 2, _C0 + w + 8, cb), _CDT),
            pltpu.SemaphoreType.DMA(()),
            pltpu.SemaphoreType.DMA(()),
        ],
        compiler_params=cparams,
    )(up_nhwc, br_nhwc, w0u, w0b,
      conv0_b.reshape(1, cout0).astype(jnp.float32))

    m = float(n * h * w)
    sc0, sh0 = _scale_shift(st0, bn0_gamma, bn0_beta, m)

    def conv_bnr(y_prev, sc, sh, wmat, bias, cin, cout):
        return pl.pallas_call(
            functools.partial(_conv_bnr_kernel, h, nt),
            out_shape=(jax.ShapeDtypeStruct((n, h, w, cout), _CDT),
                       jax.ShapeDtypeStruct((n, nt, 2, cout), jnp.float32)),
            grid=grid,
            in_specs=[
                pl.BlockSpec(memory_space=pl.ANY),
                pl.BlockSpec((9 * cin, cout), lambda i, t: (0, 0)),
                pl.BlockSpec((1, cout), lambda i, t: (0, 0)),
                pl.BlockSpec((1, 1, cin), lambda i, t: (0, 0, 0)),
                pl.BlockSpec((1, 1, cin), lambda i, t: (0, 0, 0)),
            ],
            out_specs=(
                pl.BlockSpec((1, th, w, cout), lambda i, t: (i, t, 0, 0)),
                pl.BlockSpec((1, 1, 2, cout), lambda i, t: (i, t, 0, 0)),
            ),
            scratch_shapes=[
                pltpu.VMEM((th + 2, _C0 + w + 8, cin), _CDT),
                pltpu.SemaphoreType.DMA(()),
            ],
            compiler_params=cparams,
        )(y_prev, wmat, bias.reshape(1, cout).astype(jnp.float32),
          sc.reshape(1, 1, cin), sh.reshape(1, 1, cin))

    y1, st1 = conv_bnr(y0, sc0, sh0, _wmat(conv1_w), conv1_b, cout0, cout1)
    sc1, sh1 = _scale_shift(st1, bn1_gamma, bn1_beta, m)

    y2, st2 = conv_bnr(y1, sc1, sh1, _wmat(conv2_w), conv2_b, cout1, cout2)
    sc2, sh2 = _scale_shift(st2, bn2_gamma, bn2_beta, m)

    out = pl.pallas_call(
        _bnr_out_kernel,
        out_shape=jax.ShapeDtypeStruct((n, h, w, cout2), jnp.float32),
        grid=grid,
        in_specs=[
            pl.BlockSpec((1, th, w, cout2), lambda i, t: (i, t, 0, 0)),
            pl.BlockSpec((1, 1, 1, cout2), lambda i, t: (0, 0, 0, 0)),
            pl.BlockSpec((1, 1, 1, cout2), lambda i, t: (0, 0, 0, 0)),
        ],
        out_specs=pl.BlockSpec((1, th, w, cout2), lambda i, t: (i, t, 0, 0)),
        compiler_params=cparams,
    )(y2, sc2.reshape(1, 1, 1, cout2), sh2.reshape(1, 1, 1, cout2))
    return jnp.transpose(out, (0, 3, 1, 2))
